# baseline (speedup 1.0000x reference)
import functools
import math

import jax
import jax.numpy as jnp
from jax import lax
from jax.experimental import pallas as pl
from jax.experimental.pallas import tpu as pltpu
from jax.experimental.pallas import tpu_sc as plsc

N = 10000
E = 320000
D = 128
DE = 16
H = 128

NC = 2
NS = 16
NW = NC * NS
EPT = E // NW
CH = 400
NCHUNK = EPT // CH
CHE = 160
NCHUNKE = EPT // CHE
CHT = EPT - NCHUNKE * CHE
NP = 10240
RPS = NP // NS

_mesh = lambda: plsc.VectorSubcoreMesh(core_axis_name="c", subcore_axis_name="s")


def _gelu(x):
    return x * 0.5 * (1.0 + lax.erf(x * (1.0 / math.sqrt(2.0))))


def _sc_gather_pairs(table, src, dst):
    EL = src.shape[0]
    ept = EL // NW
    nfull = ept // CH
    tail = ept - nfull * CH

    @functools.partial(
        pl.kernel,
        out_type=(jax.ShapeDtypeStruct((EL, D), jnp.float32),
                  jax.ShapeDtypeStruct((EL, D), jnp.float32)),
        mesh=_mesh(),
        compiler_params=pltpu.CompilerParams(use_tc_tiling_on_sc=False),
        scratch_types=[pltpu.VMEM((CH,), jnp.int32),
                       pltpu.VMEM((CH,), jnp.int32),
                       pltpu.VMEM((CH, D), jnp.float32),
                       pltpu.VMEM((CH, D), jnp.float32),
                       pltpu.SemaphoreType.DMA,
                       pltpu.SemaphoreType.DMA],
    )
    def k(table_hbm, src_hbm, dst_hbm, xj_hbm, xi_hbm,
          idx_s, idx_d, rows_s, rows_d, sem_s, sem_d):
        wid = lax.axis_index("s") * NC + lax.axis_index("c")
        base = wid * ept

        def do_chunk(off, n):
            isl = pl.ds(0, n)
            pltpu.sync_copy(src_hbm.at[pl.ds(off, n)], idx_s.at[isl])
            pltpu.sync_copy(dst_hbm.at[pl.ds(off, n)], idx_d.at[isl])
            ga = pltpu.async_copy(table_hbm.at[idx_s.at[isl]], rows_s.at[isl], sem_s)
            gb = pltpu.async_copy(table_hbm.at[idx_d.at[isl]], rows_d.at[isl], sem_d)
            ga.wait()
            gb.wait()
            pltpu.sync_copy(rows_s.at[isl], xj_hbm.at[pl.ds(off, n)])
            pltpu.sync_copy(rows_d.at[isl], xi_hbm.at[pl.ds(off, n)])

        @pl.loop(0, nfull)
        def _(ci):
            do_chunk(base + ci * CH, CH)

        if tail:
            do_chunk(base + nfull * CH, tail)

    return k(table, src, dst)


def _sc_scatter_sum16(vals16, src, zeros16, ones_ch):
    EL = src.shape[0]
    ept = EL // NW
    nfull = ept // CH
    tail = ept - nfull * CH

    @functools.partial(
        pl.kernel,
        out_type=(jax.ShapeDtypeStruct((NC, NP, DE), jnp.float32),
                  jax.ShapeDtypeStruct((NC, NP, DE), jnp.float32)),
        mesh=_mesh(),
        compiler_params=pltpu.CompilerParams(use_tc_tiling_on_sc=False),
        scratch_types=[pltpu.VMEM((CH,), jnp.int32),
                       pltpu.VMEM((CH, DE), jnp.float32),
                       pltpu.VMEM((CH, DE), jnp.float32),
                       pltpu.VMEM_SHARED((NP, DE), jnp.float32),
                       pltpu.VMEM_SHARED((NP, DE), jnp.float32)],
    )
    def k(v_hbm, src_hbm, z_hbm, one_hbm, sums_hbm, cnts_hbm,
          idx, vals, ones_v, acc, cnt):
        cid = lax.axis_index("c")
        sid = lax.axis_index("s")
        wid = sid * NC + cid
        rs = pl.ds(sid * RPS, RPS)
        pltpu.sync_copy(z_hbm.at[rs], acc.at[rs])
        pltpu.sync_copy(z_hbm.at[rs], cnt.at[rs])
        pltpu.sync_copy(one_hbm, ones_v)
        plsc.subcore_barrier()

        base = wid * ept

        def do_chunk(off, n):
            isl = pl.ds(0, n)
            pltpu.sync_copy(src_hbm.at[pl.ds(off, n)], idx.at[isl])
            pltpu.sync_copy(v_hbm.at[pl.ds(off, n)], vals.at[isl])
            pltpu.sync_copy(vals.at[isl], acc.at[idx.at[isl]], add=True)
            pltpu.sync_copy(ones_v.at[isl], cnt.at[idx.at[isl]], add=True)

        @pl.loop(0, nfull)
        def _(ci):
            do_chunk(base + ci * CH, CH)

        if tail:
            do_chunk(base + nfull * CH, tail)

        plsc.subcore_barrier()
        pltpu.sync_copy(acc.at[rs], sums_hbm.at[cid].at[rs])
        pltpu.sync_copy(cnt.at[rs], cnts_hbm.at[cid].at[rs])

    return k(vals16, src, zeros16, ones_ch)


def _sc_smooth(un_half, src, dst, zeros128):
    @functools.partial(
        pl.kernel,
        out_type=(jax.ShapeDtypeStruct((E, D), jnp.float32),
                  jax.ShapeDtypeStruct((NC, NP, D), jnp.float32)),
        mesh=_mesh(),
        scratch_types=[pltpu.VMEM((CHE,), jnp.int32),
                       pltpu.VMEM((CHE,), jnp.int32),
                       pltpu.VMEM((CHE, D), jnp.float32),
                       pltpu.VMEM((CHE, D), jnp.float32),
                       pltpu.VMEM_SHARED((NP, D), jnp.float32),
                       pltpu.SemaphoreType.DMA,
                       pltpu.SemaphoreType.DMA],
    )
    def k(un_hbm, src_hbm, dst_hbm, z_hbm, ue2_hbm, q_hbm,
          idx_s, idx_d, uj, ui, acc, sem_s, sem_d):
        cid = lax.axis_index("c")
        sid = lax.axis_index("s")
        wid = sid * NC + cid
        rs = pl.ds(sid * RPS, RPS)
        pltpu.sync_copy(z_hbm.at[rs], acc.at[rs])
        plsc.subcore_barrier()

        base = wid * EPT

        def do_chunk(off, n):
            isl = pl.ds(0, n)
            pltpu.sync_copy(src_hbm.at[pl.ds(off, n)], idx_s.at[isl])
            pltpu.sync_copy(dst_hbm.at[pl.ds(off, n)], idx_d.at[isl])
            ga = pltpu.async_copy(un_hbm.at[idx_s.at[isl]], uj.at[isl], sem_s)
            gb = pltpu.async_copy(un_hbm.at[idx_d.at[isl]], ui.at[isl], sem_d)
            ga.wait()
            gb.wait()

            @pl.loop(0, n)
            def _(r):
                for g in range(D // 16):
                    sl = pl.ds(g * 16, 16)
                    uj.at[r, sl][...] = ui.at[r, sl][...] + uj.at[r, sl][...]

            pltpu.sync_copy(uj.at[isl], ue2_hbm.at[pl.ds(off, n)])
            pltpu.sync_copy(uj.at[isl], acc.at[idx_s.at[isl]], add=True)

        @pl.loop(0, NCHUNKE)
        def _(ci):
            do_chunk(base + ci * CHE, CHE)

        do_chunk(base + NCHUNKE * CHE, CHT)

        plsc.subcore_barrier()
        pltpu.sync_copy(acc.at[rs], q_hbm.at[cid].at[rs])

    return k(un_half, src, dst, zeros128)


def _tc_edge_mlp(xj, xi, edge_attr, w1a, w1b, w1e, b1, w2, b2):
    EL = xj.shape[0]
    BLK = 2000
    grid = (EL // BLK,)

    def body(xj_r, xi_r, ea_r, w1a_r, w1b_r, w1e_r, b1_r, w2_r, b2_r, out_r):
        a = ((xi_r[...] + xj_r[...]) * 0.5).astype(jnp.bfloat16)
        b = (jnp.abs(xi_r[...] - xj_r[...]) * 0.5).astype(jnp.bfloat16)
        h = jnp.dot(a, w1a_r[...].astype(jnp.bfloat16),
                    preferred_element_type=jnp.float32)
        h += jnp.dot(b, w1b_r[...].astype(jnp.bfloat16),
                     preferred_element_type=jnp.float32)
        h += jnp.dot(ea_r[...].astype(jnp.bfloat16),
                     w1e_r[...].astype(jnp.bfloat16),
                     preferred_element_type=jnp.float32)
        h = _gelu(h + b1_r[...]).astype(jnp.bfloat16)
        u = jnp.dot(h, w2_r[...].astype(jnp.bfloat16),
                    preferred_element_type=jnp.float32) + b2_r[...]
        out_r[...] = _gelu(u)

    return pl.pallas_call(
        body,
        grid=grid,
        in_specs=[
            pl.BlockSpec((BLK, D), lambda i: (i, 0)),
            pl.BlockSpec((BLK, D), lambda i: (i, 0)),
            pl.BlockSpec((BLK, DE), lambda i: (i, 0)),
            pl.BlockSpec((D, H), lambda i: (0, 0)),
            pl.BlockSpec((D, H), lambda i: (0, 0)),
            pl.BlockSpec((DE, H), lambda i: (0, 0)),
            pl.BlockSpec((1, H), lambda i: (0, 0)),
            pl.BlockSpec((H, DE), lambda i: (0, 0)),
            pl.BlockSpec((1, DE), lambda i: (0, 0)),
        ],
        out_specs=pl.BlockSpec((BLK, DE), lambda i: (i, 0)),
        out_shape=jax.ShapeDtypeStruct((EL, DE), jnp.float32),
    )(xj, xi, edge_attr, w1a, w1b, w1e, b1, w2, b2)


def _tc_node_mlp(node_attr, sums, cnts, w1a, w1b, b1, w2, b2):
    BLK = 2000
    grid = (N // BLK,)

    def body(x_r, s_r, c_r, w1a_r, w1b_r, b1_r, w2_r, b2_r, out_r):
        c = jnp.maximum(c_r[0] + c_r[1] + c_r[2] + c_r[3], 1.0)
        agg = (s_r[0] + s_r[1] + s_r[2] + s_r[3]) / c
        h = jnp.dot(x_r[...], w1a_r[...], preferred_element_type=jnp.float32)
        h += jnp.dot(agg, w1b_r[...], preferred_element_type=jnp.float32)
        h = _gelu(h + b1_r[...])
        u = jnp.dot(h, w2_r[...], preferred_element_type=jnp.float32) + b2_r[...]
        out_r[...] = _gelu(u) * 0.5

    return pl.pallas_call(
        body,
        grid=grid,
        in_specs=[
            pl.BlockSpec((BLK, D), lambda i: (i, 0)),
            pl.BlockSpec((2 * NC, BLK, DE), lambda i: (0, i, 0)),
            pl.BlockSpec((2 * NC, BLK, DE), lambda i: (0, i, 0)),
            pl.BlockSpec((D, H), lambda i: (0, 0)),
            pl.BlockSpec((DE, H), lambda i: (0, 0)),
            pl.BlockSpec((1, H), lambda i: (0, 0)),
            pl.BlockSpec((H, D), lambda i: (0, 0)),
            pl.BlockSpec((1, D), lambda i: (0, 0)),
        ],
        out_specs=pl.BlockSpec((BLK, D), lambda i: (i, 0)),
        out_shape=jax.ShapeDtypeStruct((N, D), jnp.float32),
    )(node_attr, sums, cnts, w1a, w1b, b1, w2, b2)


def _tc_final_mean(q, cnts):
    BLK = 2000
    grid = (N // BLK,)

    def body(q_r, c_r, out_r):
        c = jnp.maximum(c_r[0, :, 0:1] + c_r[1, :, 0:1]
                        + c_r[2, :, 0:1] + c_r[3, :, 0:1], 1.0)
        out_r[...] = (q_r[0] + q_r[1]) / c

    return pl.pallas_call(
        body,
        grid=grid,
        in_specs=[
            pl.BlockSpec((NC, BLK, D), lambda i: (0, i, 0)),
            pl.BlockSpec((2 * NC, BLK, DE), lambda i: (0, i, 0)),
        ],
        out_specs=pl.BlockSpec((BLK, D), lambda i: (i, 0)),
        out_shape=jax.ShapeDtypeStruct((N, D), jnp.float32),
    )(q, cnts)


def kernel(node_attr, edge_idx, edge_attr, eW1, eb1, eW2, eb2,
           nW1, nb1, nW2, nb2):
    src = edge_idx[0]
    dst = edge_idx[1]
    EH = E // 2
    z16 = jnp.zeros((NP, DE), jnp.float32)
    z128 = jnp.zeros((NP, D), jnp.float32)
    ones_ch = jnp.ones((CH, DE), jnp.float32)
    eW1a, eW1b, eW1e = eW1[:D], eW1[D:2 * D], eW1[2 * D:]
    eb1r, eb2r = eb1.reshape(1, H), eb2.reshape(1, DE)

    src0, dst0 = src[:EH], dst[:EH]
    src1, dst1 = src[EH:], dst[EH:]
    xj0, xi0 = _sc_gather_pairs(node_attr, src0, dst0)
    xj1, xi1 = _sc_gather_pairs(node_attr, src1, dst1)
    ue0 = _tc_edge_mlp(xj0, xi0, edge_attr[:EH], eW1a, eW1b, eW1e, eb1r, eW2, eb2r)
    ue1 = _tc_edge_mlp(xj1, xi1, edge_attr[EH:], eW1a, eW1b, eW1e, eb1r, eW2, eb2r)
    s0, c0 = _sc_scatter_sum16(ue0, src0, z16, ones_ch)
    s1, c1 = _sc_scatter_sum16(ue1, src1, z16, ones_ch)
    sums = jnp.concatenate([s0, s1], axis=0)
    cnts = jnp.concatenate([c0, c1], axis=0)
    un_half = _tc_node_mlp(node_attr, sums, cnts,
                           nW1[:D], nW1[D:], nb1.reshape(1, H),
                           nW2, nb2.reshape(1, D))
    ue2, q = _sc_smooth(un_half, src, dst, z128)
    out_nodes = _tc_final_mean(q, cnts)
    return (out_nodes, ue2)


# speedup vs baseline: 1.0570x; 1.0570x over previous
import functools
import math

import jax
import jax.numpy as jnp
from jax import lax
from jax.experimental import pallas as pl
from jax.experimental.pallas import tpu as pltpu
from jax.experimental.pallas import tpu_sc as plsc

N = 10000
E = 320000
D = 128
DE = 16
H = 128

NC = 2
NS = 16
NW = NC * NS
EPT = E // NW
CH = 400
NCHUNK = EPT // CH
CHE = 160
NCHUNKE = EPT // CHE
CHT = EPT - NCHUNKE * CHE
NP = 10240
RPS = NP // NS

_mesh = lambda: plsc.VectorSubcoreMesh(core_axis_name="c", subcore_axis_name="s")


def _gelu(x):
    return x * 0.5 * (1.0 + lax.erf(x * (1.0 / math.sqrt(2.0))))


def _sc_gather_pairs(table, src, dst):
    @functools.partial(
        pl.kernel,
        out_type=(jax.ShapeDtypeStruct((E, D), jnp.float32),
                  jax.ShapeDtypeStruct((E, D), jnp.float32)),
        mesh=_mesh(),
        compiler_params=pltpu.CompilerParams(use_tc_tiling_on_sc=False),
        scratch_types=[pltpu.VMEM((CH,), jnp.int32),
                       pltpu.VMEM((CH,), jnp.int32),
                       pltpu.VMEM((CH, D), jnp.float32),
                       pltpu.VMEM((CH, D), jnp.float32),
                       pltpu.SemaphoreType.DMA,
                       pltpu.SemaphoreType.DMA],
    )
    def k(table_hbm, src_hbm, dst_hbm, xj_hbm, xi_hbm,
          idx_s, idx_d, rows_s, rows_d, sem_s, sem_d):
        wid = lax.axis_index("s") * NC + lax.axis_index("c")
        base = wid * EPT

        @pl.loop(0, NCHUNK)
        def _(ci):
            off = base + ci * CH
            pltpu.sync_copy(src_hbm.at[pl.ds(off, CH)], idx_s)
            pltpu.sync_copy(dst_hbm.at[pl.ds(off, CH)], idx_d)
            ga = pltpu.async_copy(table_hbm.at[idx_s], rows_s, sem_s)
            gb = pltpu.async_copy(table_hbm.at[idx_d], rows_d, sem_d)
            ga.wait()
            gb.wait()
            pltpu.sync_copy(rows_s, xj_hbm.at[pl.ds(off, CH)])
            pltpu.sync_copy(rows_d, xi_hbm.at[pl.ds(off, CH)])

    return k(table, src, dst)


def _sc_scatter_sum16(vals16, src, zeros16, ones_ch):
    @functools.partial(
        pl.kernel,
        out_type=(jax.ShapeDtypeStruct((NC, NP, DE), jnp.float32),
                  jax.ShapeDtypeStruct((NC, NP, DE), jnp.float32)),
        mesh=_mesh(),
        compiler_params=pltpu.CompilerParams(use_tc_tiling_on_sc=False),
        scratch_types=[pltpu.VMEM((CH,), jnp.int32),
                       pltpu.VMEM((CH, DE), jnp.float32),
                       pltpu.VMEM((CH, DE), jnp.float32),
                       pltpu.VMEM_SHARED((NP, DE), jnp.float32),
                       pltpu.VMEM_SHARED((NP, DE), jnp.float32)],
    )
    def k(v_hbm, src_hbm, z_hbm, one_hbm, sums_hbm, cnts_hbm,
          idx, vals, ones_v, acc, cnt):
        cid = lax.axis_index("c")
        sid = lax.axis_index("s")
        wid = sid * NC + cid
        rs = pl.ds(sid * RPS, RPS)
        pltpu.sync_copy(z_hbm.at[rs], acc.at[rs])
        pltpu.sync_copy(z_hbm.at[rs], cnt.at[rs])
        pltpu.sync_copy(one_hbm, ones_v)
        plsc.subcore_barrier()

        base = wid * EPT

        @pl.loop(0, NCHUNK)
        def _(ci):
            off = base + ci * CH
            pltpu.sync_copy(src_hbm.at[pl.ds(off, CH)], idx)
            pltpu.sync_copy(v_hbm.at[pl.ds(off, CH)], vals)
            pltpu.sync_copy(vals, acc.at[idx], add=True)
            pltpu.sync_copy(ones_v, cnt.at[idx], add=True)

        plsc.subcore_barrier()
        pltpu.sync_copy(acc.at[rs], sums_hbm.at[cid].at[rs])
        pltpu.sync_copy(cnt.at[rs], cnts_hbm.at[cid].at[rs])

    return k(vals16, src, zeros16, ones_ch)


def _sc_smooth(un_half, src, dst, zeros128):
    @functools.partial(
        pl.kernel,
        out_type=(jax.ShapeDtypeStruct((E, D), jnp.float32),
                  jax.ShapeDtypeStruct((NC, NP, D), jnp.float32)),
        mesh=_mesh(),
        scratch_types=[pltpu.VMEM((CHE,), jnp.int32),
                       pltpu.VMEM((CHE,), jnp.int32),
                       pltpu.VMEM((CHE, D), jnp.float32),
                       pltpu.VMEM((CHE, D), jnp.float32),
                       pltpu.VMEM_SHARED((NP, D), jnp.float32),
                       pltpu.SemaphoreType.DMA,
                       pltpu.SemaphoreType.DMA],
    )
    def k(un_hbm, src_hbm, dst_hbm, z_hbm, ue2_hbm, q_hbm,
          idx_s, idx_d, uj, ui, acc, sem_s, sem_d):
        cid = lax.axis_index("c")
        sid = lax.axis_index("s")
        wid = sid * NC + cid
        rs = pl.ds(sid * RPS, RPS)
        pltpu.sync_copy(z_hbm.at[rs], acc.at[rs])
        plsc.subcore_barrier()

        base = wid * EPT

        def do_chunk(off, n):
            isl = pl.ds(0, n)
            pltpu.sync_copy(src_hbm.at[pl.ds(off, n)], idx_s.at[isl])
            pltpu.sync_copy(dst_hbm.at[pl.ds(off, n)], idx_d.at[isl])
            ga = pltpu.async_copy(un_hbm.at[idx_s.at[isl]], uj.at[isl], sem_s)
            gb = pltpu.async_copy(un_hbm.at[idx_d.at[isl]], ui.at[isl], sem_d)
            ga.wait()
            gb.wait()

            @pl.loop(0, n)
            def _(r):
                for g in range(D // 16):
                    sl = pl.ds(g * 16, 16)
                    uj.at[r, sl][...] = ui.at[r, sl][...] + uj.at[r, sl][...]

            pltpu.sync_copy(uj.at[isl], ue2_hbm.at[pl.ds(off, n)])
            pltpu.sync_copy(uj.at[isl], acc.at[idx_s.at[isl]], add=True)

        @pl.loop(0, NCHUNKE)
        def _(ci):
            do_chunk(base + ci * CHE, CHE)

        do_chunk(base + NCHUNKE * CHE, CHT)

        plsc.subcore_barrier()
        pltpu.sync_copy(acc.at[rs], q_hbm.at[cid].at[rs])

    return k(un_half, src, dst, zeros128)


def _tc_edge_mlp(xj, xi, edge_attr, w1a, w1b, w1e, b1, w2, b2):
    BLK = 2560
    grid = (E // BLK,)

    def body(xj_r, xi_r, ea_r, w1a_r, w1b_r, w1e_r, b1_r, w2_r, b2_r, out_r):
        a = ((xi_r[...] + xj_r[...]) * 0.5).astype(jnp.bfloat16)
        b = (jnp.abs(xi_r[...] - xj_r[...]) * 0.5).astype(jnp.bfloat16)
        h = jnp.dot(a, w1a_r[...].astype(jnp.bfloat16),
                    preferred_element_type=jnp.float32)
        h += jnp.dot(b, w1b_r[...].astype(jnp.bfloat16),
                     preferred_element_type=jnp.float32)
        h += jnp.dot(ea_r[...].astype(jnp.bfloat16),
                     w1e_r[...].astype(jnp.bfloat16),
                     preferred_element_type=jnp.float32)
        h = _gelu(h + b1_r[...]).astype(jnp.bfloat16)
        u = jnp.dot(h, w2_r[...].astype(jnp.bfloat16),
                    preferred_element_type=jnp.float32) + b2_r[...]
        out_r[...] = _gelu(u)

    return pl.pallas_call(
        body,
        grid=grid,
        in_specs=[
            pl.BlockSpec((BLK, D), lambda i: (i, 0)),
            pl.BlockSpec((BLK, D), lambda i: (i, 0)),
            pl.BlockSpec((BLK, DE), lambda i: (i, 0)),
            pl.BlockSpec((D, H), lambda i: (0, 0)),
            pl.BlockSpec((D, H), lambda i: (0, 0)),
            pl.BlockSpec((DE, H), lambda i: (0, 0)),
            pl.BlockSpec((1, H), lambda i: (0, 0)),
            pl.BlockSpec((H, DE), lambda i: (0, 0)),
            pl.BlockSpec((1, DE), lambda i: (0, 0)),
        ],
        out_specs=pl.BlockSpec((BLK, DE), lambda i: (i, 0)),
        out_shape=jax.ShapeDtypeStruct((E, DE), jnp.float32),
    )(xj, xi, edge_attr, w1a, w1b, w1e, b1, w2, b2)


def _tc_node_mlp(node_attr, sums, cnts, w1a, w1b, b1, w2, b2):
    BLK = 2000
    grid = (N // BLK,)

    def body(x_r, s_r, c_r, w1a_r, w1b_r, b1_r, w2_r, b2_r, out_r):
        c = jnp.maximum(c_r[0] + c_r[1], 1.0)
        agg = (s_r[0] + s_r[1]) / c
        h = jnp.dot(x_r[...], w1a_r[...], preferred_element_type=jnp.float32)
        h += jnp.dot(agg, w1b_r[...], preferred_element_type=jnp.float32)
        h = _gelu(h + b1_r[...])
        u = jnp.dot(h, w2_r[...], preferred_element_type=jnp.float32) + b2_r[...]
        out_r[...] = _gelu(u) * 0.5

    return pl.pallas_call(
        body,
        grid=grid,
        in_specs=[
            pl.BlockSpec((BLK, D), lambda i: (i, 0)),
            pl.BlockSpec((NC, BLK, DE), lambda i: (0, i, 0)),
            pl.BlockSpec((NC, BLK, DE), lambda i: (0, i, 0)),
            pl.BlockSpec((D, H), lambda i: (0, 0)),
            pl.BlockSpec((DE, H), lambda i: (0, 0)),
            pl.BlockSpec((1, H), lambda i: (0, 0)),
            pl.BlockSpec((H, D), lambda i: (0, 0)),
            pl.BlockSpec((1, D), lambda i: (0, 0)),
        ],
        out_specs=pl.BlockSpec((BLK, D), lambda i: (i, 0)),
        out_shape=jax.ShapeDtypeStruct((N, D), jnp.float32),
    )(node_attr, sums, cnts, w1a, w1b, b1, w2, b2)


def _tc_final_mean(q, cnts):
    BLK = 2000
    grid = (N // BLK,)

    def body(q_r, c_r, out_r):
        c = jnp.maximum(c_r[0, :, 0:1] + c_r[1, :, 0:1], 1.0)
        out_r[...] = (q_r[0] + q_r[1]) / c

    return pl.pallas_call(
        body,
        grid=grid,
        in_specs=[
            pl.BlockSpec((NC, BLK, D), lambda i: (0, i, 0)),
            pl.BlockSpec((NC, BLK, DE), lambda i: (0, i, 0)),
        ],
        out_specs=pl.BlockSpec((BLK, D), lambda i: (i, 0)),
        out_shape=jax.ShapeDtypeStruct((N, D), jnp.float32),
    )(q, cnts)


def kernel(node_attr, edge_idx, edge_attr, eW1, eb1, eW2, eb2,
           nW1, nb1, nW2, nb2):
    src = edge_idx[0]
    dst = edge_idx[1]
    z16 = jnp.zeros((NP, DE), jnp.float32)
    z128 = jnp.zeros((NP, D), jnp.float32)
    ones_ch = jnp.ones((CH, DE), jnp.float32)

    xj, xi = _sc_gather_pairs(node_attr, src, dst)
    ue = _tc_edge_mlp(xj, xi, edge_attr,
                      eW1[:D], eW1[D:2 * D], eW1[2 * D:],
                      eb1.reshape(1, H), eW2, eb2.reshape(1, DE))
    sums, cnts = _sc_scatter_sum16(ue, src, z16, ones_ch)
    un_half = _tc_node_mlp(node_attr, sums, cnts,
                           nW1[:D], nW1[D:], nb1.reshape(1, H),
                           nW2, nb2.reshape(1, D))
    ue2, q = _sc_smooth(un_half, src, dst, z128)
    out_nodes = _tc_final_mean(q, cnts)
    return (out_nodes, ue2)
